# baseline (speedup 1.0000x reference)
import functools

import jax
import jax.numpy as jnp
from jax import lax
from jax.experimental import pallas as pl
from jax.experimental.pallas import tpu as pltpu
from jax.experimental.pallas import tpu_sc as plsc

NC = 2
NS = 16
NW = NC * NS

_BLOCK_ROWS = 10000

_INV_SQRT2 = 0.7071067811865476


def _histogram(ids_grouped, num_rows_padded):
    nwin, win = ids_grouped.shape[1], ids_grouped.shape[2]
    slc = num_rows_padded // NS
    mesh = plsc.VectorSubcoreMesh(core_axis_name="c", subcore_axis_name="s")

    @functools.partial(
        pl.kernel,
        mesh=mesh,
        out_type=jax.ShapeDtypeStruct((NC * num_rows_padded,), jnp.float32),
        scratch_types=[
            pltpu.VMEM((nwin, win), jnp.int32),
            pltpu.VMEM((win,), jnp.float32),
            pltpu.VMEM((slc,), jnp.float32),
            pltpu.VMEM_SHARED((num_rows_padded,), jnp.float32),
        ],
    )
    def hist_kernel(ids_hbm, part_hbm, ids_v, ones_v, zeros_v, counts_sh):
        cid = lax.axis_index("c")
        sid = lax.axis_index("s")
        wid = sid * NC + cid

        def init_ones(i, carry):
            ones_v[pl.ds(i * 16, 16)] = jnp.ones((16,), jnp.float32)
            return carry

        lax.fori_loop(0, win // 16, init_ones, 0)

        def init_zeros(i, carry):
            zeros_v[pl.ds(i * 16, 16)] = jnp.zeros((16,), jnp.float32)
            return carry

        lax.fori_loop(0, slc // 16, init_zeros, 0)

        pltpu.sync_copy(zeros_v, counts_sh.at[pl.ds(sid * slc, slc)])
        plsc.subcore_barrier()

        pltpu.sync_copy(ids_hbm.at[wid], ids_v)

        def scatter_window(w, carry):
            pltpu.sync_copy(ones_v, counts_sh.at[ids_v.at[w]], add=True)
            return carry

        lax.fori_loop(0, nwin, scatter_window, 0)
        plsc.subcore_barrier()

        pltpu.sync_copy(
            counts_sh.at[pl.ds(sid * slc, slc)],
            part_hbm.at[pl.ds(cid * num_rows_padded + sid * slc, slc)],
        )

    return hist_kernel(ids_grouped)


def _gather_rows(h4, ids_grouped, d):
    nch, ch = ids_grouped.shape[1], ids_grouped.shape[2]
    n_total = NW * nch * ch
    dp = h4.shape[1]
    per_w = nch * ch
    mesh = plsc.VectorSubcoreMesh(core_axis_name="c", subcore_axis_name="s")

    @functools.partial(
        pl.kernel,
        mesh=mesh,
        out_type=jax.ShapeDtypeStruct((n_total, d), jnp.float32),
        compiler_params=pltpu.CompilerParams(use_tc_tiling_on_sc=False),
        scratch_types=[
            pltpu.VMEM((nch, ch), jnp.int32),
            pltpu.VMEM((ch, dp), jnp.float32),
            pltpu.VMEM((ch, dp), jnp.float32),
            pltpu.SemaphoreType.DMA,
            pltpu.SemaphoreType.DMA,
        ],
    )
    def gather_kernel(h4_hbm, ids_hbm, out_hbm, ids_v, rows0, rows1, sem0, sem1):
        cid = lax.axis_index("c")
        sid = lax.axis_index("s")
        wid = sid * NC + cid
        base = wid * per_w

        pltpu.sync_copy(ids_hbm.at[wid], ids_v)

        pltpu.async_copy(h4_hbm.at[ids_v.at[0]], rows0, sem0)

        def pair(w, carry):
            pltpu.async_copy(h4_hbm.at[ids_v.at[w + 1]], rows1, sem1)
            pltpu.make_async_copy(h4_hbm.at[ids_v.at[w]], rows0, sem0).wait()
            pltpu.sync_copy(
                rows0.at[pl.ds(0, ch), pl.ds(0, d)],
                out_hbm.at[pl.ds(base + w * ch, ch)],
            )

            @pl.when(w + 2 < nch)
            def _():
                pltpu.async_copy(h4_hbm.at[ids_v.at[w + 2]], rows0, sem0)

            pltpu.make_async_copy(h4_hbm.at[ids_v.at[w + 1]], rows1, sem1).wait()
            pltpu.sync_copy(
                rows1.at[pl.ds(0, ch), pl.ds(0, d)],
                out_hbm.at[pl.ds(base + (w + 1) * ch, ch)],
            )
            return carry

        lax.fori_loop(0, nch // 2, lambda i, c: pair(i * 2, c), 0)

    return gather_kernel(h4, ids_grouped)


def _layer(x, p0, p1, wt, bias, scale, shift, pad_out=False):
    m, d = x.shape
    r = _BLOCK_ROWS if m % _BLOCK_ROWS == 0 else m
    nb = m // r

    def body(x_ref, p0_ref, p1_ref, w_ref, b_ref, sc_ref, sh_ref, h_ref, st_ref):
        j = pl.program_id(0)
        xb = x_ref[...] * sc_ref[...] + sh_ref[...]
        z = jnp.dot(xb, w_ref[...], preferred_element_type=jnp.float32) + b_ref[...]
        h = z * 0.5 * (1.0 + lax.erf(z * _INV_SQRT2))
        if pad_out:
            h_ref[...] = jnp.concatenate([h, jnp.zeros_like(h)], axis=1)
        else:
            h_ref[...] = h
        c = p0_ref[0] + p1_ref[0]
        s = jnp.dot(c, h, preferred_element_type=jnp.float32)
        q = jnp.dot(c, h * h, preferred_element_type=jnp.float32)
        part = jnp.concatenate([s, q], axis=0)

        @pl.when(j == 0)
        def _():
            st_ref[...] = jnp.zeros_like(st_ref)

        st_ref[...] += part

    return pl.pallas_call(
        body,
        grid=(nb,),
        in_specs=[
            pl.BlockSpec((r, d), lambda j: (j, 0)),
            pl.BlockSpec((1, 1, r), lambda j: (j, 0, 0)),
            pl.BlockSpec((1, 1, r), lambda j: (j, 0, 0)),
            pl.BlockSpec((d, d), lambda j: (0, 0)),
            pl.BlockSpec((1, d), lambda j: (0, 0)),
            pl.BlockSpec((1, d), lambda j: (0, 0)),
            pl.BlockSpec((1, d), lambda j: (0, 0)),
        ],
        out_specs=[
            pl.BlockSpec((r, 2 * d if pad_out else d), lambda j: (j, 0)),
            pl.BlockSpec((2, d), lambda j: (0, 0)),
        ],
        out_shape=[
            jax.ShapeDtypeStruct((m, 2 * d if pad_out else d), jnp.float32),
            jax.ShapeDtypeStruct((2, d), jnp.float32),
        ],
    )(x, p0, p1, wt, bias, scale, shift)


def _transpose_out(gath, bsz, csz, d):
    cd = csz * d
    bb = 512
    x = gath.reshape(bsz, cd)

    def body(x_ref, o_ref):
        o_ref[...] = x_ref[...].T

    out2d = pl.pallas_call(
        body,
        grid=(cd // 128, bsz // bb),
        in_specs=[pl.BlockSpec((bb, 128), lambda i, j: (j, i))],
        out_specs=pl.BlockSpec((128, bb), lambda i, j: (i, j)),
        out_shape=jax.ShapeDtypeStruct((cd, bsz), jnp.float32),
    )(x)
    return jnp.transpose(out2d.reshape(csz, d, bsz), (2, 0, 1))


def _bn_fold(stats, gamma, beta, n_total):
    mean = stats[0] / n_total
    var = stats[1] / n_total - mean * mean
    inv = lax.rsqrt(var + 1e-5)
    scale = gamma * inv
    shift = beta - mean * scale
    return scale[None, :], shift[None, :]


def kernel(perturbation_ids, table, W1, b1, W2, b2, W3, b3, W4, b4,
           gamma1, beta1, gamma2, beta2, gamma3, beta3):
    bsz, csz = perturbation_ids.shape
    m, d = table.shape
    n_total = bsz * csz

    ids = perturbation_ids.reshape(-1).astype(jnp.int32)

    win = 128
    nwin = n_total // (NW * win)
    ids_hist = ids.reshape(NW, nwin, win)

    pad = ((m + NS * 128 - 1) // (NS * 128)) * (NS * 128)
    partials = _histogram(ids_hist, pad)
    r = _BLOCK_ROWS if m % _BLOCK_ROWS == 0 else m
    p0 = partials[:m].reshape(m // r, 1, r)
    p1 = partials[pad:pad + m].reshape(m // r, 1, r)

    one = jnp.ones((1, d), jnp.float32)
    zero = jnp.zeros((1, d), jnp.float32)
    nf = jnp.float32(n_total)

    h1, st1 = _layer(table, p0, p1, W1.T, b1[None, :], one, zero)
    sc1, sh1 = _bn_fold(st1, gamma1, beta1, nf)
    h2, st2 = _layer(h1, p0, p1, W2.T, b2[None, :], sc1, sh1)
    sc2, sh2 = _bn_fold(st2, gamma2, beta2, nf)
    h3, st3 = _layer(h2, p0, p1, W3.T, b3[None, :], sc2, sh2)
    sc3, sh3 = _bn_fold(st3, gamma3, beta3, nf)
    h4, _ = _layer(h3, p0, p1, W4.T, b4[None, :], sc3, sh3)

    ch = 640
    nch = n_total // (NW * ch)
    out = _gather_rows(h4, ids.reshape(NW, nch, ch), d)
    return _transpose_out(out, bsz, csz, d)


# speedup vs baseline: 2.0593x; 2.0593x over previous
import functools

import jax
import jax.numpy as jnp
from jax import lax
from jax.experimental import pallas as pl
from jax.experimental.pallas import tpu as pltpu
from jax.experimental.pallas import tpu_sc as plsc

NC = 2
NS = 16
NW = NC * NS

_BLOCK_ROWS = 10000

_INV_SQRT2 = 0.7071067811865476


def _histogram(ids_grouped, num_rows_padded):
    nwin, win = ids_grouped.shape[1], ids_grouped.shape[2]
    slc = num_rows_padded // NS
    mesh = plsc.VectorSubcoreMesh(core_axis_name="c", subcore_axis_name="s")

    @functools.partial(
        pl.kernel,
        mesh=mesh,
        out_type=jax.ShapeDtypeStruct((NC * num_rows_padded,), jnp.float32),
        scratch_types=[
            pltpu.VMEM((nwin, win), jnp.int32),
            pltpu.VMEM((win,), jnp.float32),
            pltpu.VMEM((slc,), jnp.float32),
            pltpu.VMEM_SHARED((num_rows_padded,), jnp.float32),
        ],
    )
    def hist_kernel(ids_hbm, part_hbm, ids_v, ones_v, zeros_v, counts_sh):
        cid = lax.axis_index("c")
        sid = lax.axis_index("s")
        wid = sid * NC + cid

        def init_ones(i, carry):
            ones_v[pl.ds(i * 16, 16)] = jnp.ones((16,), jnp.float32)
            return carry

        lax.fori_loop(0, win // 16, init_ones, 0)

        def init_zeros(i, carry):
            zeros_v[pl.ds(i * 16, 16)] = jnp.zeros((16,), jnp.float32)
            return carry

        lax.fori_loop(0, slc // 16, init_zeros, 0)

        pltpu.sync_copy(zeros_v, counts_sh.at[pl.ds(sid * slc, slc)])
        plsc.subcore_barrier()

        pltpu.sync_copy(ids_hbm.at[wid], ids_v)

        def scatter_window(w, carry):
            pltpu.sync_copy(ones_v, counts_sh.at[ids_v.at[w]], add=True)
            return carry

        lax.fori_loop(0, nwin, scatter_window, 0)
        plsc.subcore_barrier()

        pltpu.sync_copy(
            counts_sh.at[pl.ds(sid * slc, slc)],
            part_hbm.at[pl.ds(cid * num_rows_padded + sid * slc, slc)],
        )

    return hist_kernel(ids_grouped)


def _gather_rows(h4, ids_grouped, d):
    nch, ch = ids_grouped.shape[1], ids_grouped.shape[2]
    n_total = NW * nch * ch
    dp = h4.shape[1]
    per_w = nch * ch
    mesh = plsc.VectorSubcoreMesh(core_axis_name="c", subcore_axis_name="s")

    @functools.partial(
        pl.kernel,
        mesh=mesh,
        out_type=jax.ShapeDtypeStruct((n_total, d), jnp.float32),
        compiler_params=pltpu.CompilerParams(use_tc_tiling_on_sc=False),
        scratch_types=[
            pltpu.VMEM((nch, ch), jnp.int32),
            pltpu.VMEM((ch, dp), jnp.float32),
            pltpu.VMEM((ch, dp), jnp.float32),
            pltpu.SemaphoreType.DMA,
            pltpu.SemaphoreType.DMA,
        ],
    )
    def gather_kernel(h4_hbm, ids_hbm, out_hbm, ids_v, rows0, rows1, sem0, sem1):
        cid = lax.axis_index("c")
        sid = lax.axis_index("s")
        wid = sid * NC + cid
        base = wid * per_w

        pltpu.sync_copy(ids_hbm.at[wid], ids_v)

        pltpu.async_copy(h4_hbm.at[ids_v.at[0]], rows0, sem0)

        def pair(w, carry):
            pltpu.async_copy(h4_hbm.at[ids_v.at[w + 1]], rows1, sem1)
            pltpu.make_async_copy(h4_hbm.at[ids_v.at[w]], rows0, sem0).wait()
            pltpu.sync_copy(
                rows0.at[pl.ds(0, ch), pl.ds(0, d)],
                out_hbm.at[pl.ds(base + w * ch, ch)],
            )

            @pl.when(w + 2 < nch)
            def _():
                pltpu.async_copy(h4_hbm.at[ids_v.at[w + 2]], rows0, sem0)

            pltpu.make_async_copy(h4_hbm.at[ids_v.at[w + 1]], rows1, sem1).wait()
            pltpu.sync_copy(
                rows1.at[pl.ds(0, ch), pl.ds(0, d)],
                out_hbm.at[pl.ds(base + (w + 1) * ch, ch)],
            )
            return carry

        lax.fori_loop(0, nch // 2, lambda i, c: pair(i * 2, c), 0)

    return gather_kernel(h4, ids_grouped)


def _layer(x, p0, p1, wt, bias, scale, shift, pad_out=False):
    m, d = x.shape
    r = _BLOCK_ROWS if m % _BLOCK_ROWS == 0 else m
    nb = m // r

    def body(x_ref, p0_ref, p1_ref, w_ref, b_ref, sc_ref, sh_ref, h_ref, st_ref):
        j = pl.program_id(0)
        xb = x_ref[...] * sc_ref[...] + sh_ref[...]
        z = jnp.dot(xb, w_ref[...], preferred_element_type=jnp.float32) + b_ref[...]
        h = z * 0.5 * (1.0 + lax.erf(z * _INV_SQRT2))
        if pad_out:
            h_ref[...] = jnp.concatenate([h, jnp.zeros_like(h)], axis=1)
        else:
            h_ref[...] = h
        c = p0_ref[0] + p1_ref[0]
        s = jnp.dot(c, h, preferred_element_type=jnp.float32)
        q = jnp.dot(c, h * h, preferred_element_type=jnp.float32)
        part = jnp.concatenate([s, q], axis=0)

        @pl.when(j == 0)
        def _():
            st_ref[...] = jnp.zeros_like(st_ref)

        st_ref[...] += part

    return pl.pallas_call(
        body,
        grid=(nb,),
        in_specs=[
            pl.BlockSpec((r, d), lambda j: (j, 0)),
            pl.BlockSpec((1, 1, r), lambda j: (j, 0, 0)),
            pl.BlockSpec((1, 1, r), lambda j: (j, 0, 0)),
            pl.BlockSpec((d, d), lambda j: (0, 0)),
            pl.BlockSpec((1, d), lambda j: (0, 0)),
            pl.BlockSpec((1, d), lambda j: (0, 0)),
            pl.BlockSpec((1, d), lambda j: (0, 0)),
        ],
        out_specs=[
            pl.BlockSpec((r, 2 * d if pad_out else d), lambda j: (j, 0)),
            pl.BlockSpec((2, d), lambda j: (0, 0)),
        ],
        out_shape=[
            jax.ShapeDtypeStruct((m, 2 * d if pad_out else d), jnp.float32),
            jax.ShapeDtypeStruct((2, d), jnp.float32),
        ],
    )(x, p0, p1, wt, bias, scale, shift)


def _transpose_out(gath, bsz, csz, d):
    cd = csz * d
    k = cd // 128
    bb = 128
    x = gath.reshape(bsz * k, 128)

    def body(x_ref, o_ref):
        xb = x_ref[...]
        x3 = xb.reshape(bb, k, 128)
        cols = [x3[:, i, :].T for i in range(k)]
        o_ref[...] = jnp.concatenate(cols, axis=0)

    out2d = pl.pallas_call(
        body,
        grid=(bsz // bb,),
        in_specs=[pl.BlockSpec((bb * k, 128), lambda j: (j, 0))],
        out_specs=pl.BlockSpec((cd, bb), lambda j: (0, j)),
        out_shape=jax.ShapeDtypeStruct((cd, bsz), jnp.float32),
    )(x)
    return jnp.transpose(out2d.reshape(csz, d, bsz), (2, 0, 1))


def _bn_fold(stats, gamma, beta, n_total):
    mean = stats[0] / n_total
    var = stats[1] / n_total - mean * mean
    inv = lax.rsqrt(var + 1e-5)
    scale = gamma * inv
    shift = beta - mean * scale
    return scale[None, :], shift[None, :]


def kernel(perturbation_ids, table, W1, b1, W2, b2, W3, b3, W4, b4,
           gamma1, beta1, gamma2, beta2, gamma3, beta3):
    bsz, csz = perturbation_ids.shape
    m, d = table.shape
    n_total = bsz * csz

    ids = perturbation_ids.reshape(-1).astype(jnp.int32)

    win = 128
    nwin = n_total // (NW * win)
    ids_hist = ids.reshape(NW, nwin, win)

    pad = ((m + NS * 128 - 1) // (NS * 128)) * (NS * 128)
    partials = _histogram(ids_hist, pad)
    r = _BLOCK_ROWS if m % _BLOCK_ROWS == 0 else m
    p0 = partials[:m].reshape(m // r, 1, r)
    p1 = partials[pad:pad + m].reshape(m // r, 1, r)

    one = jnp.ones((1, d), jnp.float32)
    zero = jnp.zeros((1, d), jnp.float32)
    nf = jnp.float32(n_total)

    h1, st1 = _layer(table, p0, p1, W1.T, b1[None, :], one, zero)
    sc1, sh1 = _bn_fold(st1, gamma1, beta1, nf)
    h2, st2 = _layer(h1, p0, p1, W2.T, b2[None, :], sc1, sh1)
    sc2, sh2 = _bn_fold(st2, gamma2, beta2, nf)
    h3, st3 = _layer(h2, p0, p1, W3.T, b3[None, :], sc2, sh2)
    sc3, sh3 = _bn_fold(st3, gamma3, beta3, nf)
    h4, _ = _layer(h3, p0, p1, W4.T, b4[None, :], sc3, sh3)

    ch = 640
    nch = n_total // (NW * ch)
    out = _gather_rows(h4, ids.reshape(NW, nch, ch), d)
    return _transpose_out(out, bsz, csz, d)


# speedup vs baseline: 2.1219x; 1.0304x over previous
import functools

import jax
import jax.numpy as jnp
from jax import lax
from jax.experimental import pallas as pl
from jax.experimental.pallas import tpu as pltpu
from jax.experimental.pallas import tpu_sc as plsc

NC = 2
NS = 16
NW = NC * NS

_BLOCK_ROWS = 10000

_INV_SQRT2 = 0.7071067811865476


def _histogram(ids_grouped, num_rows_padded):
    nwin, win = ids_grouped.shape[1], ids_grouped.shape[2]
    slc = num_rows_padded // NS
    mesh = plsc.VectorSubcoreMesh(core_axis_name="c", subcore_axis_name="s")

    @functools.partial(
        pl.kernel,
        mesh=mesh,
        out_type=jax.ShapeDtypeStruct((NC * num_rows_padded,), jnp.float32),
        scratch_types=[
            pltpu.VMEM((nwin, win), jnp.int32),
            pltpu.VMEM((win,), jnp.float32),
            pltpu.VMEM((slc,), jnp.float32),
            pltpu.VMEM_SHARED((num_rows_padded,), jnp.float32),
        ],
    )
    def hist_kernel(ids_hbm, part_hbm, ids_v, ones_v, zeros_v, counts_sh):
        cid = lax.axis_index("c")
        sid = lax.axis_index("s")
        wid = sid * NC + cid

        def init_ones(i, carry):
            ones_v[pl.ds(i * 16, 16)] = jnp.ones((16,), jnp.float32)
            return carry

        lax.fori_loop(0, win // 16, init_ones, 0)

        def init_zeros(i, carry):
            zeros_v[pl.ds(i * 16, 16)] = jnp.zeros((16,), jnp.float32)
            return carry

        lax.fori_loop(0, slc // 16, init_zeros, 0)

        pltpu.sync_copy(zeros_v, counts_sh.at[pl.ds(sid * slc, slc)])
        plsc.subcore_barrier()

        pltpu.sync_copy(ids_hbm.at[wid], ids_v)

        def scatter_window(w, carry):
            pltpu.sync_copy(ones_v, counts_sh.at[ids_v.at[w]], add=True)
            return carry

        lax.fori_loop(0, nwin, scatter_window, 0)
        plsc.subcore_barrier()

        pltpu.sync_copy(
            counts_sh.at[pl.ds(sid * slc, slc)],
            part_hbm.at[pl.ds(cid * num_rows_padded + sid * slc, slc)],
        )

    return hist_kernel(ids_grouped)


def _gather_rows(h4, ids_grouped, d):
    nch, ch = ids_grouped.shape[1], ids_grouped.shape[2]
    n_total = NW * nch * ch
    dp = h4.shape[1]
    per_w = nch * ch
    mesh = plsc.VectorSubcoreMesh(core_axis_name="c", subcore_axis_name="s")

    @functools.partial(
        pl.kernel,
        mesh=mesh,
        out_type=jax.ShapeDtypeStruct((n_total, d), jnp.float32),
        compiler_params=pltpu.CompilerParams(use_tc_tiling_on_sc=False),
        scratch_types=[
            pltpu.VMEM((nch, ch), jnp.int32),
            pltpu.VMEM((ch, dp), jnp.float32),
            pltpu.VMEM((ch, dp), jnp.float32),
            pltpu.SemaphoreType.DMA,
            pltpu.SemaphoreType.DMA,
        ],
    )
    def gather_kernel(h4_hbm, ids_hbm, out_hbm, ids_v, rows0, rows1, sem0, sem1):
        cid = lax.axis_index("c")
        sid = lax.axis_index("s")
        wid = sid * NC + cid
        base = wid * per_w

        pltpu.sync_copy(ids_hbm.at[wid], ids_v)

        pltpu.async_copy(h4_hbm.at[ids_v.at[0]], rows0, sem0)

        def pair(w, carry):
            pltpu.async_copy(h4_hbm.at[ids_v.at[w + 1]], rows1, sem1)
            pltpu.make_async_copy(h4_hbm.at[ids_v.at[w]], rows0, sem0).wait()
            pltpu.sync_copy(
                rows0.at[pl.ds(0, ch), pl.ds(0, d)],
                out_hbm.at[pl.ds(base + w * ch, ch)],
            )

            @pl.when(w + 2 < nch)
            def _():
                pltpu.async_copy(h4_hbm.at[ids_v.at[w + 2]], rows0, sem0)

            pltpu.make_async_copy(h4_hbm.at[ids_v.at[w + 1]], rows1, sem1).wait()
            pltpu.sync_copy(
                rows1.at[pl.ds(0, ch), pl.ds(0, d)],
                out_hbm.at[pl.ds(base + (w + 1) * ch, ch)],
            )
            return carry

        lax.fori_loop(0, nch // 2, lambda i, c: pair(i * 2, c), 0)

    return gather_kernel(h4, ids_grouped)


def _layer(x, p0, p1, wt, bias, scale, shift, pad_out=False):
    m, d = x.shape
    r = _BLOCK_ROWS if m % _BLOCK_ROWS == 0 else m
    nb = m // r

    def body(x_ref, p0_ref, p1_ref, w_ref, b_ref, sc_ref, sh_ref, h_ref, st_ref):
        j = pl.program_id(0)
        xb = x_ref[...] * sc_ref[...] + sh_ref[...]
        z = jnp.dot(xb, w_ref[...], preferred_element_type=jnp.float32) + b_ref[...]
        h = z * 0.5 * (1.0 + lax.erf(z * _INV_SQRT2))
        if pad_out:
            h_ref[...] = jnp.concatenate([h, jnp.zeros_like(h)], axis=1)
        else:
            h_ref[...] = h
        c = p0_ref[0] + p1_ref[0]
        s = jnp.dot(c, h, preferred_element_type=jnp.float32)
        q = jnp.dot(c, h * h, preferred_element_type=jnp.float32)
        part = jnp.concatenate([s, q], axis=0)

        @pl.when(j == 0)
        def _():
            st_ref[...] = jnp.zeros_like(st_ref)

        st_ref[...] += part

    return pl.pallas_call(
        body,
        grid=(nb,),
        in_specs=[
            pl.BlockSpec((r, d), lambda j: (j, 0)),
            pl.BlockSpec((1, 1, r), lambda j: (j, 0, 0)),
            pl.BlockSpec((1, 1, r), lambda j: (j, 0, 0)),
            pl.BlockSpec((d, d), lambda j: (0, 0)),
            pl.BlockSpec((1, d), lambda j: (0, 0)),
            pl.BlockSpec((1, d), lambda j: (0, 0)),
            pl.BlockSpec((1, d), lambda j: (0, 0)),
        ],
        out_specs=[
            pl.BlockSpec((r, 2 * d if pad_out else d), lambda j: (j, 0)),
            pl.BlockSpec((2, d), lambda j: (0, 0)),
        ],
        out_shape=[
            jax.ShapeDtypeStruct((m, 2 * d if pad_out else d), jnp.float32),
            jax.ShapeDtypeStruct((2, d), jnp.float32),
        ],
    )(x, p0, p1, wt, bias, scale, shift)


def _mlp_fused(table, p0, p1, wstack, bstack, gstack, bestack, n_total):
    m, d = table.shape
    r = _BLOCK_ROWS if m % _BLOCK_ROWS == 0 else m
    nb = m // r
    nf = float(n_total)

    def body(tab_ref, p0_ref, p1_ref, w_ref, b_ref, g_ref, be_ref, h4_ref,
             hs, acc, scale_s, shift_s, sem):
        p = pl.program_id(0)
        j = pl.program_id(1)

        @pl.when(jnp.logical_and(p == 0, j == 0))
        def _():
            scale_s[...] = jnp.ones_like(scale_s)
            shift_s[...] = jnp.zeros_like(shift_s)

        @pl.when(p == 0)
        def _():
            pltpu.make_async_copy(
                tab_ref.at[pl.ds(j * r, r)], hs.at[pl.ds(j * r, r)], sem
            ).start()
            pltpu.make_async_copy(
                tab_ref.at[pl.ds(j * r, r)], hs.at[pl.ds(j * r, r)], sem
            ).wait()

        @pl.when(jnp.logical_and(p > 0, j == 0))
        def _():
            mean = acc[0:1, :] / nf
            var = acc[1:2, :] / nf - mean * mean
            inv = lax.rsqrt(var + 1e-5)
            scale_s[...] = g_ref[0] * inv
            shift_s[...] = be_ref[0] - mean * scale_s[...]

        xb = hs[pl.ds(j * r, r), :] * scale_s[...] + shift_s[...]
        z = jnp.dot(xb, w_ref[0], preferred_element_type=jnp.float32) + b_ref[0]
        h = z * 0.5 * (1.0 + lax.erf(z * _INV_SQRT2))
        hs[pl.ds(j * r, r), :] = h

        c = p0_ref[0] + p1_ref[0]
        s = jnp.dot(c, h, preferred_element_type=jnp.float32)
        q = jnp.dot(c, h * h, preferred_element_type=jnp.float32)
        part = jnp.concatenate([s, q], axis=0)

        @pl.when(j == 0)
        def _():
            acc[...] = jnp.zeros_like(acc)

        acc[...] += part

        @pl.when(p == 3)
        def _():
            pltpu.make_async_copy(
                hs.at[pl.ds(j * r, r)], h4_ref.at[pl.ds(j * r, r)], sem
            ).start()
            pltpu.make_async_copy(
                hs.at[pl.ds(j * r, r)], h4_ref.at[pl.ds(j * r, r)], sem
            ).wait()

    return pl.pallas_call(
        body,
        grid=(4, nb),
        in_specs=[
            pl.BlockSpec(memory_space=pltpu.MemorySpace.HBM),
            pl.BlockSpec((1, 1, r), lambda p, j: (j, 0, 0)),
            pl.BlockSpec((1, 1, r), lambda p, j: (j, 0, 0)),
            pl.BlockSpec((1, d, d), lambda p, j: (p, 0, 0)),
            pl.BlockSpec((1, 1, d), lambda p, j: (p, 0, 0)),
            pl.BlockSpec((1, 1, d), lambda p, j: (jnp.maximum(p, 1) - 1, 0, 0)),
            pl.BlockSpec((1, 1, d), lambda p, j: (jnp.maximum(p, 1) - 1, 0, 0)),
        ],
        out_specs=pl.BlockSpec(memory_space=pltpu.MemorySpace.HBM),
        out_shape=jax.ShapeDtypeStruct((m, d), jnp.float32),
        scratch_shapes=[
            pltpu.VMEM((m, d), jnp.float32),
            pltpu.VMEM((2, d), jnp.float32),
            pltpu.VMEM((1, d), jnp.float32),
            pltpu.VMEM((1, d), jnp.float32),
            pltpu.SemaphoreType.DMA,
        ],
    )(table, p0, p1, wstack, bstack, gstack, bestack)


def _transpose_out(gath, bsz, csz, d):
    cd = csz * d
    k = cd // 128
    bb = 128
    x = gath.reshape(bsz * k, 128)

    def body(x_ref, o_ref):
        xb = x_ref[...]
        x3 = xb.reshape(bb, k, 128)
        cols = [x3[:, i, :].T for i in range(k)]
        o_ref[...] = jnp.concatenate(cols, axis=0)

    out2d = pl.pallas_call(
        body,
        grid=(bsz // bb,),
        in_specs=[pl.BlockSpec((bb * k, 128), lambda j: (j, 0))],
        out_specs=pl.BlockSpec((cd, bb), lambda j: (0, j)),
        out_shape=jax.ShapeDtypeStruct((cd, bsz), jnp.float32),
    )(x)
    return jnp.transpose(out2d.reshape(csz, d, bsz), (2, 0, 1))


def _bn_fold(stats, gamma, beta, n_total):
    mean = stats[0] / n_total
    var = stats[1] / n_total - mean * mean
    inv = lax.rsqrt(var + 1e-5)
    scale = gamma * inv
    shift = beta - mean * scale
    return scale[None, :], shift[None, :]


def kernel(perturbation_ids, table, W1, b1, W2, b2, W3, b3, W4, b4,
           gamma1, beta1, gamma2, beta2, gamma3, beta3):
    bsz, csz = perturbation_ids.shape
    m, d = table.shape
    n_total = bsz * csz

    ids = perturbation_ids.reshape(-1).astype(jnp.int32)

    win = 128
    nwin = n_total // (NW * win)
    ids_hist = ids.reshape(NW, nwin, win)

    pad = ((m + NS * 128 - 1) // (NS * 128)) * (NS * 128)
    partials = _histogram(ids_hist, pad)
    r = _BLOCK_ROWS if m % _BLOCK_ROWS == 0 else m
    p0 = partials[:m].reshape(m // r, 1, r)
    p1 = partials[pad:pad + m].reshape(m // r, 1, r)

    one = jnp.ones((1, d), jnp.float32)
    zero = jnp.zeros((1, d), jnp.float32)
    nf = jnp.float32(n_total)

    wstack = jnp.stack([W1.T, W2.T, W3.T, W4.T])
    bstack = jnp.stack([b1, b2, b3, b4])[:, None, :]
    gstack = jnp.stack([gamma1, gamma2, gamma3])[:, None, :]
    bestack = jnp.stack([beta1, beta2, beta3])[:, None, :]
    h4 = _mlp_fused(table, p0, p1, wstack, bstack, gstack, bestack, n_total)

    ch = 640
    nch = n_total // (NW * ch)
    out = _gather_rows(h4, ids.reshape(NW, nch, ch), d)
    return _transpose_out(out, bsz, csz, d)


# speedup vs baseline: 2.1267x; 1.0023x over previous
import functools

import jax
import jax.numpy as jnp
from jax import lax
from jax.experimental import pallas as pl
from jax.experimental.pallas import tpu as pltpu
from jax.experimental.pallas import tpu_sc as plsc

NC = 2
NS = 16
NW = NC * NS

_BLOCK_ROWS = 10000

_INV_SQRT2 = 0.7071067811865476


def _histogram(ids_grouped, num_rows_padded):
    nwin, win = ids_grouped.shape[1], ids_grouped.shape[2]
    slc = num_rows_padded // NS
    mesh = plsc.VectorSubcoreMesh(core_axis_name="c", subcore_axis_name="s")

    @functools.partial(
        pl.kernel,
        mesh=mesh,
        out_type=jax.ShapeDtypeStruct((NC * num_rows_padded,), jnp.float32),
        scratch_types=[
            pltpu.VMEM((nwin, win), jnp.int32),
            pltpu.VMEM((win,), jnp.float32),
            pltpu.VMEM((slc,), jnp.float32),
            pltpu.VMEM_SHARED((num_rows_padded,), jnp.float32),
        ],
    )
    def hist_kernel(ids_hbm, part_hbm, ids_v, ones_v, zeros_v, counts_sh):
        cid = lax.axis_index("c")
        sid = lax.axis_index("s")
        wid = sid * NC + cid

        def init_ones(i, carry):
            ones_v[pl.ds(i * 16, 16)] = jnp.ones((16,), jnp.float32)
            return carry

        lax.fori_loop(0, win // 16, init_ones, 0)

        def init_zeros(i, carry):
            zeros_v[pl.ds(i * 16, 16)] = jnp.zeros((16,), jnp.float32)
            return carry

        lax.fori_loop(0, slc // 16, init_zeros, 0)

        pltpu.sync_copy(zeros_v, counts_sh.at[pl.ds(sid * slc, slc)])
        plsc.subcore_barrier()

        pltpu.sync_copy(ids_hbm.at[wid], ids_v)

        def scatter_window(w, carry):
            pltpu.sync_copy(ones_v, counts_sh.at[ids_v.at[w]], add=True)
            return carry

        lax.fori_loop(0, nwin, scatter_window, 0)
        plsc.subcore_barrier()

        pltpu.sync_copy(
            counts_sh.at[pl.ds(sid * slc, slc)],
            part_hbm.at[pl.ds(cid * num_rows_padded + sid * slc, slc)],
        )

    return hist_kernel(ids_grouped)


def _gather_rows(h4, ids_grouped, d):
    nch, ch = ids_grouped.shape[1], ids_grouped.shape[2]
    n_total = NW * nch * ch
    dp = h4.shape[1]
    per_w = nch * ch
    mesh = plsc.VectorSubcoreMesh(core_axis_name="c", subcore_axis_name="s")

    @functools.partial(
        pl.kernel,
        mesh=mesh,
        out_type=jax.ShapeDtypeStruct((n_total, d), jnp.float32),
        compiler_params=pltpu.CompilerParams(use_tc_tiling_on_sc=False),
        scratch_types=[
            pltpu.VMEM((nch, ch), jnp.int32),
            pltpu.VMEM((ch, dp), jnp.float32),
            pltpu.VMEM((ch, dp), jnp.float32),
            pltpu.SemaphoreType.DMA,
            pltpu.SemaphoreType.DMA,
        ],
    )
    def gather_kernel(h4_hbm, ids_hbm, out_hbm, ids_v, rows0, rows1, sem0, sem1):
        cid = lax.axis_index("c")
        sid = lax.axis_index("s")
        wid = sid * NC + cid
        base = wid * per_w

        pltpu.sync_copy(ids_hbm.at[wid], ids_v)

        pltpu.async_copy(h4_hbm.at[ids_v.at[0]], rows0, sem0)

        def pair(w, carry):
            pltpu.async_copy(h4_hbm.at[ids_v.at[w + 1]], rows1, sem1)
            pltpu.make_async_copy(h4_hbm.at[ids_v.at[w]], rows0, sem0).wait()
            pltpu.sync_copy(
                rows0.at[pl.ds(0, ch), pl.ds(0, d)],
                out_hbm.at[pl.ds(base + w * ch, ch)],
            )

            @pl.when(w + 2 < nch)
            def _():
                pltpu.async_copy(h4_hbm.at[ids_v.at[w + 2]], rows0, sem0)

            pltpu.make_async_copy(h4_hbm.at[ids_v.at[w + 1]], rows1, sem1).wait()
            pltpu.sync_copy(
                rows1.at[pl.ds(0, ch), pl.ds(0, d)],
                out_hbm.at[pl.ds(base + (w + 1) * ch, ch)],
            )
            return carry

        lax.fori_loop(0, nch // 2, lambda i, c: pair(i * 2, c), 0)

    return gather_kernel(h4, ids_grouped)


def _mlp_fused(table, p0, p1, wstack, bstack, gstack, bestack, n_total):
    m, d = table.shape
    r = _BLOCK_ROWS if m % _BLOCK_ROWS == 0 else m
    nb = m // r
    nf = float(n_total)

    def body(tab_ref, p0_ref, p1_ref, w_ref, b_ref, g_ref, be_ref, h4_ref,
             hs, acc, scale_s, shift_s, sem):
        p = pl.program_id(0)
        j = pl.program_id(1)

        @pl.when(jnp.logical_and(p == 0, j == 0))
        def _():
            scale_s[...] = jnp.ones_like(scale_s)
            shift_s[...] = jnp.zeros_like(shift_s)

        @pl.when(p == 0)
        def _():
            pltpu.make_async_copy(
                tab_ref.at[pl.ds(j * r, r)], hs.at[pl.ds(j * r, r)], sem
            ).start()
            pltpu.make_async_copy(
                tab_ref.at[pl.ds(j * r, r)], hs.at[pl.ds(j * r, r)], sem
            ).wait()

        @pl.when(jnp.logical_and(p > 0, j == 0))
        def _():
            mean = acc[0:1, :] / nf
            var = acc[1:2, :] / nf - mean * mean
            inv = lax.rsqrt(var + 1e-5)
            scale_s[...] = g_ref[0] * inv
            shift_s[...] = be_ref[0] - mean * scale_s[...]

        xb = hs[pl.ds(j * r, r), :] * scale_s[...] + shift_s[...]
        z = jnp.dot(xb, w_ref[0], preferred_element_type=jnp.float32) + b_ref[0]
        h = z * 0.5 * (1.0 + lax.erf(z * _INV_SQRT2))
        hs[pl.ds(j * r, r), :] = h

        c = p0_ref[0] + p1_ref[0]
        s = jnp.dot(c, h, preferred_element_type=jnp.float32)
        q = jnp.dot(c, h * h, preferred_element_type=jnp.float32)
        part = jnp.concatenate([s, q], axis=0)

        @pl.when(j == 0)
        def _():
            acc[...] = jnp.zeros_like(acc)

        acc[...] += part

        @pl.when(p == 3)
        def _():
            pltpu.make_async_copy(
                hs.at[pl.ds(j * r, r)], h4_ref.at[pl.ds(j * r, r)], sem
            ).start()
            pltpu.make_async_copy(
                hs.at[pl.ds(j * r, r)], h4_ref.at[pl.ds(j * r, r)], sem
            ).wait()

    return pl.pallas_call(
        body,
        grid=(4, nb),
        in_specs=[
            pl.BlockSpec(memory_space=pltpu.MemorySpace.HBM),
            pl.BlockSpec((1, 1, r), lambda p, j: (j, 0, 0)),
            pl.BlockSpec((1, 1, r), lambda p, j: (j, 0, 0)),
            pl.BlockSpec((1, d, d), lambda p, j: (p, 0, 0)),
            pl.BlockSpec((1, 1, d), lambda p, j: (p, 0, 0)),
            pl.BlockSpec((1, 1, d), lambda p, j: (jnp.maximum(p, 1) - 1, 0, 0)),
            pl.BlockSpec((1, 1, d), lambda p, j: (jnp.maximum(p, 1) - 1, 0, 0)),
        ],
        out_specs=pl.BlockSpec(memory_space=pltpu.MemorySpace.HBM),
        out_shape=jax.ShapeDtypeStruct((m, d), jnp.float32),
        scratch_shapes=[
            pltpu.VMEM((m, d), jnp.float32),
            pltpu.VMEM((2, d), jnp.float32),
            pltpu.VMEM((1, d), jnp.float32),
            pltpu.VMEM((1, d), jnp.float32),
            pltpu.SemaphoreType.DMA,
        ],
    )(table, p0, p1, wstack, bstack, gstack, bestack)


def _transpose_out(gath, bsz, csz, d):
    cd = csz * d
    k = cd // 128
    bb = 128
    x = gath.reshape(bsz * k, 128)

    def body(x_ref, o_ref):
        xb = x_ref[...]
        x3 = xb.reshape(bb, k, 128)
        cols = [x3[:, i, :].T for i in range(k)]
        o_ref[...] = jnp.concatenate(cols, axis=0)

    out2d = pl.pallas_call(
        body,
        grid=(bsz // bb,),
        in_specs=[pl.BlockSpec((bb * k, 128), lambda j: (j, 0))],
        out_specs=pl.BlockSpec((cd, bb), lambda j: (0, j)),
        out_shape=jax.ShapeDtypeStruct((cd, bsz), jnp.float32),
    )(x)
    return jnp.transpose(out2d.reshape(csz, d, bsz), (2, 0, 1))


def kernel(perturbation_ids, table, W1, b1, W2, b2, W3, b3, W4, b4,
           gamma1, beta1, gamma2, beta2, gamma3, beta3):
    bsz, csz = perturbation_ids.shape
    m, d = table.shape
    n_total = bsz * csz

    ids = perturbation_ids.reshape(-1).astype(jnp.int32)

    win = 128
    nwin = n_total // (NW * win)
    ids_hist = ids.reshape(NW, nwin, win)

    pad = ((m + NS * 128 - 1) // (NS * 128)) * (NS * 128)
    partials = _histogram(ids_hist, pad)
    r = _BLOCK_ROWS if m % _BLOCK_ROWS == 0 else m
    p0 = partials[:m].reshape(m // r, 1, r)
    p1 = partials[pad:pad + m].reshape(m // r, 1, r)

    wstack = jnp.stack([W1.T, W2.T, W3.T, W4.T])
    bstack = jnp.stack([b1, b2, b3, b4])[:, None, :]
    gstack = jnp.stack([gamma1, gamma2, gamma3])[:, None, :]
    bestack = jnp.stack([beta1, beta2, beta3])[:, None, :]
    h4 = _mlp_fused(table, p0, p1, wstack, bstack, gstack, bestack, n_total)

    ch = 640
    nch = n_total // (NW * ch)
    out = _gather_rows(h4, ids.reshape(NW, nch, ch), d)
    return _transpose_out(out, bsz, csz, d)
